# Initial kernel scaffold; baseline (speedup 1.0000x reference)
import jax
import jax.numpy as jnp
from jax.experimental import pallas as pl


def kernel(user_id, item_id, neg_item_id, P, Q):
    raise NotImplementedError("write your pallas kernel here")


# speedup vs baseline: 5.9757x; 5.9757x over previous
import functools

import jax
import jax.numpy as jnp
from jax import lax
from jax.experimental import pallas as pl
from jax.experimental.pallas import tpu as pltpu
from jax.experimental.pallas import tpu_sc as plsc

_N_USER = 100000
_N_ITEM = 100000
_D = 128
_DA = 144
_B = 4096
_K = 200
_KH = _K // 2
_MARGIN = 0.5
_LAMB_C = 10.0

_NC = 2
_NS = 16
_NW = _NC * _NS
_BPW = _B // _NW

_PREP_ROWS = 2000


def _prep_body(p_ref, q_ref, pa_ref, qa_ref):
    for src, dst in ((p_ref, pa_ref), (q_ref, qa_ref)):
        x = src[...]
        n2 = jnp.sum(x * x, axis=1, keepdims=True)
        scale = jnp.minimum(1.0, lax.rsqrt(jnp.maximum(n2, 1e-24)))
        xc = x * scale
        n2c = jnp.minimum(n2, 1.0)
        pad = jnp.zeros((x.shape[0], _DA - _D - 1), jnp.float32)
        dst[...] = jnp.concatenate([xc, n2c, pad], axis=1)


def _prep(P, Q):
    nb = _N_USER // _PREP_ROWS
    return pl.pallas_call(
        _prep_body,
        grid=(nb,),
        in_specs=[
            pl.BlockSpec((_PREP_ROWS, _D), lambda i: (i, 0)),
            pl.BlockSpec((_PREP_ROWS, _D), lambda i: (i, 0)),
        ],
        out_specs=[
            pl.BlockSpec((_PREP_ROWS, _DA), lambda i: (i, 0)),
            pl.BlockSpec((_PREP_ROWS, _DA), lambda i: (i, 0)),
        ],
        out_shape=[
            jax.ShapeDtypeStruct((_N_USER, _DA), jnp.float32),
            jax.ShapeDtypeStruct((_N_ITEM, _DA), jnp.float32),
        ],
    )(P, Q)


def _row_vregs(ref, b):
    return [ref[b, pl.ds(16 * j, 16)] for j in range(_D // 16)]


def _dot8(a, bvs):
    acc = a[0] * bvs[0]
    for j in range(1, _D // 16):
        acc = acc + a[j] * bvs[j]
    return jnp.sum(acc)


@functools.partial(
    pl.kernel,
    out_type=(
        jax.ShapeDtypeStruct((_B, 16), jnp.float32),
        jax.ShapeDtypeStruct((_B, _DA), jnp.float32),
        jax.ShapeDtypeStruct((_B, _DA), jnp.float32),
    ),
    mesh=plsc.VectorSubcoreMesh(
        core_axis_name="c", subcore_axis_name="s",
        num_cores=_NC, num_subcores=_NS,
    ),
    compiler_params=pltpu.CompilerParams(
        use_tc_tiling_on_sc=False, needs_layout_passes=False),
    scratch_types=[
        pltpu.VMEM((_BPW,), jnp.int32),
        pltpu.VMEM((_BPW,), jnp.int32),
        pltpu.VMEM((2 * _BPW, _KH), jnp.int32),
        pltpu.VMEM((_BPW, _DA), jnp.float32),
        pltpu.VMEM((_BPW, _DA), jnp.float32),
        pltpu.VMEM((2, _K, _DA), jnp.float32),
        pltpu.VMEM((_BPW, 16), jnp.float32),
        pltpu.SemaphoreType.DMA,
        pltpu.SemaphoreType.DMA,
        pltpu.SemaphoreType.DMA,
    ],
)
def _sc_main(uid_h, iid_h, neg2_h, pa_h, qa_h,
             sc_h, uc_h, itc_h,
             uid_v, iid_v, ids_v, u_v, it_v, neg_v, sc_v,
             sem_ui, sem_a, sem_b):
    wid = lax.axis_index("s") * _NC + lax.axis_index("c")
    base = wid * _BPW

    pltpu.sync_copy(uid_h.at[pl.ds(base, _BPW)], uid_v)
    pltpu.sync_copy(iid_h.at[pl.ds(base, _BPW)], iid_v)
    pltpu.sync_copy(neg2_h.at[pl.ds(2 * base, 2 * _BPW), :], ids_v)

    cu = pltpu.async_copy(pa_h.at[uid_v], u_v, sem_ui)
    ci = pltpu.async_copy(qa_h.at[iid_v], it_v, sem_ui)

    def _issue(b, slot, sem):
        pltpu.async_copy(qa_h.at[ids_v.at[2 * b]],
                         neg_v.at[slot, pl.ds(0, _KH)], sem)
        pltpu.async_copy(qa_h.at[ids_v.at[2 * b + 1]],
                         neg_v.at[slot, pl.ds(_KH, _KH)], sem)

    def _wait_full(slot, sem):
        pltpu.make_async_copy(qa_h.at[pl.ds(0, _K)], neg_v.at[slot], sem).wait()

    _issue(0, 0, sem_a)
    cu.wait()
    ci.wait()

    lane = lax.iota(jnp.int32, 16)

    def _compute(b, slot):
        u8 = _row_vregs(u_v, b)
        un2 = u_v[b, pl.ds(_D, 16)][0]
        it8 = _row_vregs(it_v, b)
        itn2 = it_v[b, pl.ds(_D, 16)][0]
        d_ij = un2 + itn2 - 2.0 * _dot8(u8, it8)
        md = _MARGIN + d_ij - un2

        def _kbody(k, carry):
            s, c = carry
            for r in range(4):
                kk = 4 * k + r
                q8 = _row_vregs(neg_v.at[slot], kk)
                qn2 = neg_v[slot, kk, pl.ds(_D, 16)][0]
                t = md - qn2 + 2.0 * _dot8(u8, q8)
                s = s + jnp.maximum(t, 0.0)
                c = c + jnp.where(t <= 0.0, 1.0, 0.0)
            return s, c

        s, c = lax.fori_loop(0, _K // 4, _kbody, (jnp.float32(0.0), jnp.float32(0.0)))
        sc_v[b, :] = jnp.where(lane == 0, s, jnp.where(lane == 1, c, 0.0))

    def _gbody(g, carry):
        b0 = 2 * g
        _issue(b0 + 1, 1, sem_b)
        _wait_full(0, sem_a)
        _compute(b0, 0)
        b_next = jnp.minimum(b0 + 2, _BPW - 1)
        _issue(b_next, 0, sem_a)
        _wait_full(1, sem_b)
        _compute(b0 + 1, 1)
        return carry

    lax.fori_loop(0, _BPW // 2, _gbody, 0)
    _wait_full(0, sem_a)

    pltpu.sync_copy(sc_v, sc_h.at[pl.ds(base, _BPW), :])
    pltpu.sync_copy(u_v, uc_h.at[pl.ds(base, _BPW), :])
    pltpu.sync_copy(it_v, itc_h.at[pl.ds(base, _BPW), :])


def _final_body(sc_ref, uc_ref, itc_ref, out_ref):
    s = sc_ref[:, 0:1]
    c = sc_ref[:, 1:2]
    w = jnp.log(c * float(_N_ITEM) / float(_K) + 1.0)
    loss_m = jnp.sum(w * s)

    u = uc_ref[:, :_D]
    it = itc_ref[:, :_D]
    m = jnp.concatenate([u, it], axis=0)
    y = m - jnp.mean(m, axis=0, keepdims=True)
    cov = lax.dot_general(y, y, (((0,), (0,)), ((), ()))) / float(2 * _B)
    fro = jnp.sqrt(jnp.sum(cov * cov))
    ii = lax.broadcasted_iota(jnp.int32, (_D, _D), 0)
    jj = lax.broadcasted_iota(jnp.int32, (_D, _D), 1)
    diag = jnp.where(ii == jj, cov, 0.0)
    diagsq = jnp.sum(diag * diag)
    cov_loss = (fro - diagsq) / float(_N_USER)
    out_ref[...] = jnp.full((1, 1), loss_m + _LAMB_C * cov_loss, jnp.float32)


def _final(sc, uc, itc):
    return pl.pallas_call(
        _final_body,
        out_shape=jax.ShapeDtypeStruct((1, 1), jnp.float32),
    )(sc, uc, itc)


def kernel(user_id, item_id, neg_item_id, P, Q):
    Pa, Qa = _prep(P, Q)
    neg2 = neg_item_id.astype(jnp.int32).reshape(2 * _B, _KH)
    sc, uc, itc = _sc_main(
        user_id.astype(jnp.int32), item_id.astype(jnp.int32), neg2, Pa, Qa)
    out = _final(sc, uc, itc)
    return out[0, 0]


# speedup vs baseline: 7.0364x; 1.1775x over previous
import functools

import jax
import jax.numpy as jnp
from jax import lax
from jax.experimental import pallas as pl
from jax.experimental.pallas import tpu as pltpu
from jax.experimental.pallas import tpu_sc as plsc

_N_USER = 100000
_N_ITEM = 100000
_D = 128
_B = 4096
_K = 200
_KH = _K // 2
_MARGIN = 0.5
_LAMB_C = 10.0

_NC = 2
_NS = 16
_NW = _NC * _NS
_BPW = _B // _NW
_RPW = _N_USER // _NW
_CHUNK = 125
_NCHUNK = _RPW // _CHUNK

_SC_PARAMS = pltpu.CompilerParams(
    use_tc_tiling_on_sc=False, needs_layout_passes=False)


def _row_vregs(ref, b):
    return [ref[b, pl.ds(16 * j, 16)] for j in range(_D // 16)]


def _dot8(a, bvs):
    acc = a[0] * bvs[0]
    for j in range(1, _D // 16):
        acc = acc + a[j] * bvs[j]
    return jnp.sum(acc)


def _rsqrt_nr(nv):
    i = plsc.bitcast(nv, jnp.int32)
    i = jnp.full((16,), 0x5F3759DF, jnp.int32) - lax.shift_right_logical(i, 1)
    y = plsc.bitcast(i, jnp.float32)
    half = 0.5 * nv
    for _ in range(3):
        y = y * (1.5 - half * y * y)
    return y


@functools.partial(
    pl.kernel,
    out_type=(
        jax.ShapeDtypeStruct((_N_USER, 16), jnp.float32),
        jax.ShapeDtypeStruct((_N_ITEM, 16), jnp.float32),
    ),
    mesh=plsc.VectorSubcoreMesh(
        core_axis_name="c", subcore_axis_name="s",
        num_cores=_NC, num_subcores=_NS,
    ),
    compiler_params=_SC_PARAMS,
    scratch_types=[
        pltpu.VMEM((2, _CHUNK, _D), jnp.float32),
        pltpu.VMEM((_RPW, 16), jnp.float32),
        pltpu.SemaphoreType.DMA,
        pltpu.SemaphoreType.DMA,
    ],
)
def _sc_prep(p_h, q_h, pmeta_h, qmeta_h, rows_v, meta_v, sem_a, sem_b):
    wid = lax.axis_index("s") * _NC + lax.axis_index("c")
    base = wid * _RPW
    lane = lax.iota(jnp.int32, 16)

    for src_h, dst_h in ((p_h, pmeta_h), (q_h, qmeta_h)):
        def _issue(c, slot, sem):
            pltpu.async_copy(
                src_h.at[pl.ds(base + c * _CHUNK, _CHUNK), :],
                rows_v.at[slot], sem)

        def _wait(slot, sem):
            pltpu.make_async_copy(
                src_h.at[pl.ds(0, _CHUNK), :], rows_v.at[slot], sem).wait()

        def _compute(c, slot):
            def _rbody(r, carry):
                q8 = _row_vregs(rows_v.at[slot], r)
                n2 = _dot8(q8, q8)
                nv = jnp.full((16,), n2, jnp.float32)
                scale = jnp.minimum(1.0, _rsqrt_nr(nv))
                n2c = jnp.minimum(nv, 1.0)
                meta = jnp.where(lane == 0, n2c,
                                 jnp.where(lane == 1, scale, 0.0))
                meta_v[c * _CHUNK + r, :] = meta
                return carry

            lax.fori_loop(0, _CHUNK, _rbody, 0)

        _issue(0, 0, sem_a)

        def _cbody(g, carry):
            c0 = 2 * g
            _issue(c0 + 1, 1, sem_b)
            _wait(0, sem_a)
            _compute(c0, 0)
            c_next = jnp.minimum(c0 + 2, _NCHUNK - 1)
            _issue(c_next, 0, sem_a)
            _wait(1, sem_b)
            _compute(c0 + 1, 1)
            return carry

        lax.fori_loop(0, _NCHUNK // 2, _cbody, 0)
        _wait(0, sem_a)
        _compute(_NCHUNK - 1, 0)

        pltpu.sync_copy(meta_v, dst_h.at[pl.ds(base, _RPW), :])


@functools.partial(
    pl.kernel,
    out_type=(
        jax.ShapeDtypeStruct((_B, 16), jnp.float32),
        jax.ShapeDtypeStruct((_B, _D), jnp.float32),
        jax.ShapeDtypeStruct((_B, _D), jnp.float32),
    ),
    mesh=plsc.VectorSubcoreMesh(
        core_axis_name="c", subcore_axis_name="s",
        num_cores=_NC, num_subcores=_NS,
    ),
    compiler_params=_SC_PARAMS,
    scratch_types=[
        pltpu.VMEM((_BPW,), jnp.int32),
        pltpu.VMEM((_BPW,), jnp.int32),
        pltpu.VMEM((2 * _BPW, _KH), jnp.int32),
        pltpu.VMEM((_BPW, _D), jnp.float32),
        pltpu.VMEM((_BPW, _D), jnp.float32),
        pltpu.VMEM((_BPW, 16), jnp.float32),
        pltpu.VMEM((_BPW, 16), jnp.float32),
        pltpu.VMEM((2, _K, _D), jnp.float32),
        pltpu.VMEM((2, _K, 16), jnp.float32),
        pltpu.VMEM((_BPW, 16), jnp.float32),
        pltpu.SemaphoreType.DMA,
        pltpu.SemaphoreType.DMA,
        pltpu.SemaphoreType.DMA,
    ],
)
def _sc_main(uid_h, iid_h, neg2_h, p_h, q_h, pmeta_h, qmeta_h,
             sc_h, uc_h, itc_h,
             uid_v, iid_v, ids_v, u_v, it_v, umeta_v, itmeta_v,
             neg_v, negmeta_v, sc_v, sem_ui, sem_a, sem_b):
    wid = lax.axis_index("s") * _NC + lax.axis_index("c")
    base = wid * _BPW

    pltpu.sync_copy(uid_h.at[pl.ds(base, _BPW)], uid_v)
    pltpu.sync_copy(iid_h.at[pl.ds(base, _BPW)], iid_v)
    pltpu.sync_copy(neg2_h.at[pl.ds(2 * base, 2 * _BPW), :], ids_v)

    cu = pltpu.async_copy(p_h.at[uid_v], u_v, sem_ui)
    cum = pltpu.async_copy(pmeta_h.at[uid_v], umeta_v, sem_ui)
    ci = pltpu.async_copy(q_h.at[iid_v], it_v, sem_ui)
    cim = pltpu.async_copy(qmeta_h.at[iid_v], itmeta_v, sem_ui)

    def _issue(b, slot, sem):
        pltpu.async_copy(q_h.at[ids_v.at[2 * b]],
                         neg_v.at[slot, pl.ds(0, _KH)], sem)
        pltpu.async_copy(q_h.at[ids_v.at[2 * b + 1]],
                         neg_v.at[slot, pl.ds(_KH, _KH)], sem)
        pltpu.async_copy(qmeta_h.at[ids_v.at[2 * b]],
                         negmeta_v.at[slot, pl.ds(0, _KH)], sem)
        pltpu.async_copy(qmeta_h.at[ids_v.at[2 * b + 1]],
                         negmeta_v.at[slot, pl.ds(_KH, _KH)], sem)

    def _wait_full(slot, sem):
        pltpu.make_async_copy(q_h.at[pl.ds(0, _K)], neg_v.at[slot], sem).wait()
        pltpu.make_async_copy(
            qmeta_h.at[pl.ds(0, _K)], negmeta_v.at[slot], sem).wait()

    _issue(0, 0, sem_a)
    cu.wait()
    cum.wait()
    ci.wait()
    cim.wait()

    lane = lax.iota(jnp.int32, 16)

    def _compute(b, slot):
        um = umeta_v[b, pl.ds(0, 16)]
        un2c = um[0]
        uscale = um[1]
        im = itmeta_v[b, pl.ds(0, 16)]
        itn2c = im[0]
        itscale = im[1]
        u8 = [uscale * v for v in _row_vregs(u_v, b)]
        it8 = [itscale * v for v in _row_vregs(it_v, b)]
        for j in range(_D // 16):
            u_v[b, pl.ds(16 * j, 16)] = u8[j]
            it_v[b, pl.ds(16 * j, 16)] = it8[j]
        d_ij = un2c + itn2c - 2.0 * _dot8(u8, it8)
        md = _MARGIN + d_ij - un2c

        def _kbody(k, carry):
            s, c = carry
            for r in range(4):
                kk = 4 * k + r
                q8 = _row_vregs(neg_v.at[slot], kk)
                qm = negmeta_v[slot, kk, pl.ds(0, 16)]
                t = md - qm[0] + (2.0 * qm[1]) * _dot8(u8, q8)
                s = s + jnp.maximum(t, 0.0)
                c = c + jnp.where(t <= 0.0, 1.0, 0.0)
            return s, c

        s, c = lax.fori_loop(0, _K // 4, _kbody,
                             (jnp.float32(0.0), jnp.float32(0.0)))
        sc_v[b, :] = jnp.where(lane == 0, s, jnp.where(lane == 1, c, 0.0))

    def _gbody(g, carry):
        b0 = 2 * g
        _issue(b0 + 1, 1, sem_b)
        _wait_full(0, sem_a)
        _compute(b0, 0)
        b_next = jnp.minimum(b0 + 2, _BPW - 1)
        _issue(b_next, 0, sem_a)
        _wait_full(1, sem_b)
        _compute(b0 + 1, 1)
        return carry

    lax.fori_loop(0, _BPW // 2, _gbody, 0)
    _wait_full(0, sem_a)

    pltpu.sync_copy(sc_v, sc_h.at[pl.ds(base, _BPW), :])
    pltpu.sync_copy(u_v, uc_h.at[pl.ds(base, _BPW), :])
    pltpu.sync_copy(it_v, itc_h.at[pl.ds(base, _BPW), :])


def _final_body(sc_ref, uc_ref, itc_ref, out_ref):
    s = sc_ref[:, 0:1]
    c = sc_ref[:, 1:2]
    w = jnp.log(c * float(_N_ITEM) / float(_K) + 1.0)
    loss_m = jnp.sum(w * s)

    m = jnp.concatenate([uc_ref[...], itc_ref[...]], axis=0)
    y = m - jnp.mean(m, axis=0, keepdims=True)
    cov = lax.dot_general(y, y, (((0,), (0,)), ((), ()))) / float(2 * _B)
    fro = jnp.sqrt(jnp.sum(cov * cov))
    ii = lax.broadcasted_iota(jnp.int32, (_D, _D), 0)
    jj = lax.broadcasted_iota(jnp.int32, (_D, _D), 1)
    diag = jnp.where(ii == jj, cov, 0.0)
    diagsq = jnp.sum(diag * diag)
    cov_loss = (fro - diagsq) / float(_N_USER)
    out_ref[...] = jnp.full((1, 1), loss_m + _LAMB_C * cov_loss, jnp.float32)


def _final(sc, uc, itc):
    return pl.pallas_call(
        _final_body,
        out_shape=jax.ShapeDtypeStruct((1, 1), jnp.float32),
    )(sc, uc, itc)


def kernel(user_id, item_id, neg_item_id, P, Q):
    pmeta, qmeta = _sc_prep(P, Q)
    neg2 = neg_item_id.astype(jnp.int32).reshape(2 * _B, _KH)
    sc, uc, itc = _sc_main(
        user_id.astype(jnp.int32), item_id.astype(jnp.int32), neg2,
        P, Q, pmeta, qmeta)
    out = _final(sc, uc, itc)
    return out[0, 0]
